# baseline (speedup 1.0000x reference)
import functools

import jax
import jax.numpy as jnp
from jax import lax
from jax.experimental import pallas as pl
from jax.experimental.pallas import tpu as pltpu
from jax.experimental.pallas import tpu_sc as plsc

_WD = 0.01
_NC, _NS, _L = 2, 16, 16
_NW = _NC * _NS
_B = 16384
_D = 64
_V = 100000
_DPW = _D // _NW
_S = 2048
_NCH = _B // _S


def _sc_body(u_hbm, i_hbm, j_hbm, wt_hbm, ht_hbm, x_hbm, reg_hbm,
             row_v, wu_v, idx1_v, idx2_v, prod_v, reg_v, sem):
    wid = lax.axis_index("s") * _NC + lax.axis_index("c")

    def dim_pass(p, reg_acc):
        d = wid * _DPW + p

        pltpu.sync_copy(wt_hbm.at[d], row_v)

        def chunk_a(c, acc):
            pltpu.sync_copy(u_hbm.at[pl.ds(c * _S, _S)], idx1_v)

            def ga(t, a):
                uvec = idx1_v[pl.ds(t * _L, _L)]
                vals = plsc.load_gather(row_v, [uvec])
                wu_v[pl.ds(c * _S + t * _L, _L)] = vals
                return a + vals * vals

            return lax.fori_loop(0, _S // _L, ga, acc)

        reg_acc = lax.fori_loop(0, _NCH, chunk_a, reg_acc)

        pltpu.sync_copy(ht_hbm.at[d], row_v)

        def chunk_b(c, acc):
            pltpu.sync_copy(i_hbm.at[pl.ds(c * _S, _S)], idx1_v)
            pltpu.sync_copy(j_hbm.at[pl.ds(c * _S, _S)], idx2_v)

            def gb(t, a):
                ivec = idx1_v[pl.ds(t * _L, _L)]
                jvec = idx2_v[pl.ds(t * _L, _L)]
                hi = plsc.load_gather(row_v, [ivec])
                hj = plsc.load_gather(row_v, [jvec])
                wu = wu_v[pl.ds(c * _S + t * _L, _L)]
                prod_v[pl.ds(t * _L, _L)] = wu * (hi - hj)
                return a + hi * hi + hj * hj

            acc = lax.fori_loop(0, _S // _L, gb, acc)
            pltpu.sync_copy(prod_v, x_hbm.at[d, pl.ds(c * _S, _S)])
            return acc

        return lax.fori_loop(0, _NCH, chunk_b, reg_acc)

    reg_acc = jnp.zeros((_L,), jnp.float32)
    for p in range(_DPW):
        reg_acc = dim_pass(p, reg_acc)

    zeros = jnp.zeros((_L,), jnp.float32)
    for r in range(8):
        for s in range(128 // _L):
            reg_v[r, pl.ds(s * _L, _L)] = zeros
    reg_v[0, pl.ds(0, _L)] = reg_acc
    pltpu.sync_copy(reg_v, reg_hbm.at[pl.ds(wid * 8, 8)])


def _sc_call(u, i, j, Wt, Ht):
    mesh = plsc.VectorSubcoreMesh(core_axis_name="c", subcore_axis_name="s")
    return pl.kernel(
        _sc_body,
        out_type=(
            jax.ShapeDtypeStruct((_D, _B), jnp.float32),
            jax.ShapeDtypeStruct((_NW * 8, 128), jnp.float32),
        ),
        mesh=mesh,
        scratch_types=[
            pltpu.VMEM((_V,), jnp.float32),
            pltpu.VMEM((_B,), jnp.float32),
            pltpu.VMEM((_S,), jnp.int32),
            pltpu.VMEM((_S,), jnp.int32),
            pltpu.VMEM((_S,), jnp.float32),
            pltpu.VMEM((8, 128), jnp.float32),
            pltpu.SemaphoreType.DMA,
        ],
        compiler_params=pltpu.CompilerParams(use_tc_tiling_on_sc=True,
                                             needs_layout_passes=False),
    )(u, i, j, Wt, Ht)


_TCB = 1024


def _tc_body(x_ref, reg_ref, o_ref):
    step = pl.program_id(0)
    x = jnp.sum(x_ref[...], axis=0)
    ls = jnp.minimum(x, 0.0) - jnp.log1p(jnp.exp(-jnp.abs(x)))
    partial = -jnp.sum(ls)

    @pl.when(step == 0)
    def _():
        o_ref[...] = jnp.broadcast_to(_WD * jnp.sum(reg_ref[...]), (1, 1))

    o_ref[...] += jnp.broadcast_to(partial, (1, 1))


def _tc_call(x, reg):
    return pl.pallas_call(
        _tc_body,
        grid=(_B // _TCB,),
        in_specs=[
            pl.BlockSpec((_D, _TCB), lambda c: (0, c)),
            pl.BlockSpec((_NW * 8, 128), lambda c: (0, 0)),
        ],
        out_specs=pl.BlockSpec((1, 1), lambda c: (0, 0)),
        out_shape=jax.ShapeDtypeStruct((1, 1), jnp.float32),
    )(x, reg)


def kernel(u, i, j, W, H):
    u = u.astype(jnp.int32)
    i = i.astype(jnp.int32)
    j = j.astype(jnp.int32)
    x, reg = _sc_call(u, i, j, W.T, H.T)
    out = _tc_call(x, reg)
    return out[0, 0]
